# baseline (speedup 1.0000x reference)
import functools
import math

import jax
import jax.numpy as jnp
import numpy as np
from jax.experimental import pallas as pl

SR = 24000
N_FFT = 1024
HOP = 320
N_MELS = 128
VOCAB = 1024
NCB = 4
DM = 512
NFRAMES = 751
YROWS = 754
HP = jax.lax.Precision.HIGHEST
DP = jax.lax.Precision.DEFAULT


def _mel_fb_np():
    n_freqs = N_FFT // 2 + 1
    all_freqs = np.linspace(0.0, SR / 2.0, n_freqs)

    def hz_to_mel(f):
        return 2595.0 * np.log10(1.0 + f / 700.0)

    def mel_to_hz(m):
        return 700.0 * (10.0 ** (m / 2595.0) - 1.0)

    m_pts = np.linspace(hz_to_mel(0.0), hz_to_mel(SR / 2.0), N_MELS + 2)
    f_pts = mel_to_hz(m_pts)
    f_diff = f_pts[1:] - f_pts[:-1]
    slopes = f_pts[None, :] - all_freqs[:, None]
    down = -slopes[:, :-2] / f_diff[:-1]
    up = slopes[:, 2:] / f_diff[1:]
    return np.maximum(0.0, np.minimum(down, up))


NFREQ = N_FFT // 2 + 1
FPAD = 640
NH = N_FFT // 2
HHOP = HOP // 2
HYROWS = 754


@functools.lru_cache(maxsize=1)
def _dft_constants():
    n = np.arange(N_FFT)
    win = 0.5 - 0.5 * np.cos(2.0 * np.pi * n / N_FFT)
    m = np.arange(NH)
    f = np.arange(NH // 2 + 1)
    fs = np.arange(1, NH // 2)
    dft = np.concatenate(
        [np.cos(2.0 * np.pi * m[:, None] * f[None, :] / NH),
         np.sin(2.0 * np.pi * m[:, None] * fs[None, :] / NH)], axis=1
    )
    ge = win[0::2][:, None] * dft
    go = win[1::2][:, None] * dft
    fq = np.arange(NH // 2)
    tw = np.stack([np.cos(2.0 * np.pi * fq / N_FFT),
                   np.sin(2.0 * np.pi * fq / N_FFT)], axis=0)
    fb = _mel_fb_np()
    w = np.zeros((FPAD, N_MELS))
    w[0:256] = fb[0:256]
    w[256:512] = fb[512:256:-1]
    w[512] = fb[256]
    return (np.asarray(ge, np.float32), np.asarray(go, np.float32),
            np.asarray(tw, np.float32), np.asarray(w, np.float32))


def _frames_half(y):
    return jnp.concatenate(
        [y[0:NFRAMES], y[1 : NFRAMES + 1], y[2 : NFRAMES + 2],
         y[3 : NFRAMES + 3, : NH - 3 * HHOP]],
        axis=1,
    )


def _tokenizer_kernel(ye_ref, yo_ref, ge_ref, go_ref, tw_ref, w_ref, a1_ref,
                      b1_ref, a2_ref, b2_ref, cbt_ref, out_ref):
    ue = jnp.dot(_frames_half(ye_ref[0]), ge_ref[...], precision=HP,
                 preferred_element_type=jnp.float32)
    uo = jnp.dot(_frames_half(yo_ref[0]), go_ref[...], precision=HP,
                 preferred_element_type=jnp.float32)
    nq = NH // 2
    z1 = jnp.zeros((NFRAMES, 1), jnp.float32)
    re_e = ue[:, :nq]
    re_o = uo[:, :nq]
    s_e = jnp.concatenate([z1, ue[:, nq + 1 :]], axis=1)
    s_o = jnp.concatenate([z1, uo[:, nq + 1 :]], axis=1)
    c = tw_ref[0:1, :]
    s = tw_ref[1:2, :]
    re_t = c * re_o - s * s_o
    im_t = -(c * s_o + s * re_o)
    im_e = -s_e
    p_plus = (re_e + re_t) ** 2 + (im_e + im_t) ** 2
    p_minus = (re_e - re_t) ** 2 + (im_e - im_t) ** 2
    p256 = ue[:, nq : nq + 1] ** 2 + uo[:, nq : nq + 1] ** 2
    power = jnp.concatenate(
        [p_plus, p_minus, p256, jnp.zeros((NFRAMES, FPAD - 2 * nq - 1),
                                          jnp.float32)], axis=1)
    mel = jnp.dot(power, w_ref[...], precision=DP,
                  preferred_element_type=jnp.float32)
    mel = jnp.log(jnp.clip(mel, 1e-5, None))

    zc = jnp.zeros((1, N_MELS), jnp.float32)
    melp = jnp.concatenate([zc, mel, zc], axis=0)
    a1 = a1_ref[...]
    h = (jnp.dot(melp[0:NFRAMES], a1[0:128], precision=DP,
                 preferred_element_type=jnp.float32)
         + jnp.dot(melp[1 : NFRAMES + 1], a1[128:256], precision=DP,
                   preferred_element_type=jnp.float32)
         + jnp.dot(melp[2 : NFRAMES + 2], a1[256:384], precision=DP,
                   preferred_element_type=jnp.float32)
         + b1_ref[...])
    h = 0.5 * h * (1.0 + jax.lax.erf(h * (1.0 / math.sqrt(2.0))))

    zh = jnp.zeros((1, 256), jnp.float32)
    hp = jnp.concatenate([zh, h, zh], axis=0)
    a2 = a2_ref[...]
    f = (jnp.dot(hp[0:NFRAMES], a2[0:256], precision=DP,
                 preferred_element_type=jnp.float32)
         + jnp.dot(hp[1 : NFRAMES + 1], a2[256:512], precision=DP,
                   preferred_element_type=jnp.float32)
         + jnp.dot(hp[2 : NFRAMES + 2], a2[512:768], precision=DP,
                   preferred_element_type=jnp.float32)
         + b2_ref[...])

    d = DM // NCB
    idx = jax.lax.broadcasted_iota(jnp.int32, (NFRAMES, VOCAB), 1)
    toks = []
    for i in range(NCB):
        cbt = cbt_ref[i]
        cn = jnp.sum(cbt * cbt, axis=0, keepdims=True)
        s = jnp.dot(f[:, i * d : (i + 1) * d], cbt, precision=DP,
                    preferred_element_type=jnp.float32)
        scores = cn - 2.0 * s
        m = jnp.min(scores, axis=-1, keepdims=True)
        toks.append(jnp.min(jnp.where(scores == m, idx, VOCAB), axis=-1)
                    .astype(jnp.int32))
    out_ref[0] = jnp.stack(toks, axis=0)


def kernel(waveform, W1, b1, W2, b2, codebooks):
    B = waveform.shape[0]
    ge_np, go_np, tw_np, w_np = _dft_constants()
    ge = jnp.asarray(ge_np)
    go = jnp.asarray(go_np)
    tw = jnp.asarray(tw_np)
    w = jnp.asarray(w_np)

    pad = N_FFT // 2
    xp = jnp.pad(waveform, ((0, 0), (pad, pad)), mode='reflect')
    ye = xp[:, 0::2]
    yo = xp[:, 1::2]
    ye = jnp.pad(ye, ((0, 0), (0, HYROWS * HHOP - ye.shape[1])))
    yo = jnp.pad(yo, ((0, 0), (0, HYROWS * HHOP - yo.shape[1])))
    ye = ye.reshape(B, HYROWS, HHOP)
    yo = yo.reshape(B, HYROWS, HHOP)

    a1 = jnp.concatenate([W1[:, :, k].T for k in range(3)], axis=0)
    a2 = jnp.concatenate([W2[:, :, k].T for k in range(3)], axis=0)
    b1r = b1.reshape(1, -1)
    b2r = b2.reshape(1, -1)
    cbt = jnp.transpose(codebooks, (0, 2, 1))

    const = lambda shape: pl.BlockSpec(shape, lambda b: (0,) * len(shape))
    out = pl.pallas_call(
        _tokenizer_kernel,
        grid=(B,),
        in_specs=[
            pl.BlockSpec((1, HYROWS, HHOP), lambda b: (b, 0, 0)),
            pl.BlockSpec((1, HYROWS, HHOP), lambda b: (b, 0, 0)),
            const((NH, NH)),
            const((NH, NH)),
            const((2, NH // 2)),
            const((FPAD, N_MELS)),
            const((384, 256)),
            const((1, 256)),
            const((768, 512)),
            const((1, 512)),
            const((NCB, DM // NCB, VOCAB)),
        ],
        out_specs=pl.BlockSpec((1, NCB, NFRAMES), lambda b: (b, 0, 0)),
        out_shape=jax.ShapeDtypeStruct((B, NCB, NFRAMES), jnp.int32),
    )(ye, yo, ge, go, tw, w, a1, b1r, a2, b2r, cbt)
    return out


# speedup vs baseline: 2.9128x; 2.9128x over previous
import functools
import math

import jax
import jax.numpy as jnp
import numpy as np
from jax.experimental import pallas as pl

SR = 24000
N_FFT = 1024
HOP = 320
N_MELS = 128
VOCAB = 1024
NCB = 4
DM = 512
NFRAMES = 751
YROWS = 754
HP = jax.lax.Precision.HIGHEST
DP = jax.lax.Precision.DEFAULT


def _mel_fb_np():
    n_freqs = N_FFT // 2 + 1
    all_freqs = np.linspace(0.0, SR / 2.0, n_freqs)

    def hz_to_mel(f):
        return 2595.0 * np.log10(1.0 + f / 700.0)

    def mel_to_hz(m):
        return 700.0 * (10.0 ** (m / 2595.0) - 1.0)

    m_pts = np.linspace(hz_to_mel(0.0), hz_to_mel(SR / 2.0), N_MELS + 2)
    f_pts = mel_to_hz(m_pts)
    f_diff = f_pts[1:] - f_pts[:-1]
    slopes = f_pts[None, :] - all_freqs[:, None]
    down = -slopes[:, :-2] / f_diff[:-1]
    up = slopes[:, 2:] / f_diff[1:]
    return np.maximum(0.0, np.minimum(down, up))


NFREQ = N_FFT // 2 + 1
FPAD = 640
NH = N_FFT // 2
HHOP = HOP // 2
HYROWS = 754


@functools.lru_cache(maxsize=1)
def _dft_constants():
    n = np.arange(N_FFT)
    win = 0.5 - 0.5 * np.cos(2.0 * np.pi * n / N_FFT)
    m = np.arange(NH)
    f = np.arange(NH // 2 + 1)
    fs = np.arange(1, NH // 2)
    dft = np.concatenate(
        [np.cos(2.0 * np.pi * m[:, None] * f[None, :] / NH),
         np.sin(2.0 * np.pi * m[:, None] * fs[None, :] / NH)], axis=1
    )
    ge = win[0::2][:, None] * dft
    go = win[1::2][:, None] * dft
    fq = np.arange(NH // 2)
    tw = np.stack([np.cos(2.0 * np.pi * fq / N_FFT),
                   np.sin(2.0 * np.pi * fq / N_FFT)], axis=0)
    fb = _mel_fb_np()
    w = np.zeros((FPAD, N_MELS))
    w[0:256] = fb[0:256]
    w[256:512] = fb[512:256:-1]
    w[512] = fb[256]
    perm = np.zeros((HOP, HOP), np.float32)
    for mm in range(HHOP):
        perm[2 * mm, mm] = 1.0
        perm[2 * mm + 1, HHOP + mm] = 1.0
    return (np.asarray(ge, np.float32), np.asarray(go, np.float32),
            np.asarray(tw, np.float32), np.asarray(w, np.float32), perm)


def _frames_half(y):
    return jnp.concatenate(
        [y[0:NFRAMES], y[1 : NFRAMES + 1], y[2 : NFRAMES + 2],
         y[3 : NFRAMES + 3, : NH - 3 * HHOP]],
        axis=1,
    )


def _tokenizer_kernel(y_ref, ge_ref, go_ref, tw_ref, w_ref, perm_ref, a1_ref,
                      b1_ref, a2_ref, b2_ref, cbt_ref, out_ref):
    y = y_ref[0]
    z = jnp.dot(y, perm_ref[...], precision=HP,
                preferred_element_type=jnp.float32)
    ye = z[:, :HHOP]
    yo = z[:, HHOP:]
    ue = jnp.dot(_frames_half(ye), ge_ref[...], precision=HP,
                 preferred_element_type=jnp.float32)
    uo = jnp.dot(_frames_half(yo), go_ref[...], precision=HP,
                 preferred_element_type=jnp.float32)
    nq = NH // 2
    z1 = jnp.zeros((NFRAMES, 1), jnp.float32)
    re_e = ue[:, :nq]
    re_o = uo[:, :nq]
    s_e = jnp.concatenate([z1, ue[:, nq + 1 :]], axis=1)
    s_o = jnp.concatenate([z1, uo[:, nq + 1 :]], axis=1)
    c = tw_ref[0:1, :]
    s = tw_ref[1:2, :]
    re_t = c * re_o - s * s_o
    im_t = -(c * s_o + s * re_o)
    im_e = -s_e
    p_plus = (re_e + re_t) ** 2 + (im_e + im_t) ** 2
    p_minus = (re_e - re_t) ** 2 + (im_e - im_t) ** 2
    p256 = ue[:, nq : nq + 1] ** 2 + uo[:, nq : nq + 1] ** 2
    power = jnp.concatenate(
        [p_plus, p_minus, p256, jnp.zeros((NFRAMES, FPAD - 2 * nq - 1),
                                          jnp.float32)], axis=1)
    mel = jnp.dot(power, w_ref[...], precision=DP,
                  preferred_element_type=jnp.float32)
    mel = jnp.log(jnp.clip(mel, 1e-5, None))

    zc = jnp.zeros((1, N_MELS), jnp.float32)
    melp = jnp.concatenate([zc, mel, zc], axis=0)
    a1 = a1_ref[...]
    h = (jnp.dot(melp[0:NFRAMES], a1[0:128], precision=DP,
                 preferred_element_type=jnp.float32)
         + jnp.dot(melp[1 : NFRAMES + 1], a1[128:256], precision=DP,
                   preferred_element_type=jnp.float32)
         + jnp.dot(melp[2 : NFRAMES + 2], a1[256:384], precision=DP,
                   preferred_element_type=jnp.float32)
         + b1_ref[...])
    h = 0.5 * h * (1.0 + jax.lax.erf(h * (1.0 / math.sqrt(2.0))))

    zh = jnp.zeros((1, 256), jnp.float32)
    hp = jnp.concatenate([zh, h, zh], axis=0)
    a2 = a2_ref[...]
    f = (jnp.dot(hp[0:NFRAMES], a2[0:256], precision=DP,
                 preferred_element_type=jnp.float32)
         + jnp.dot(hp[1 : NFRAMES + 1], a2[256:512], precision=DP,
                   preferred_element_type=jnp.float32)
         + jnp.dot(hp[2 : NFRAMES + 2], a2[512:768], precision=DP,
                   preferred_element_type=jnp.float32)
         + b2_ref[...])

    d = DM // NCB
    idx = jax.lax.broadcasted_iota(jnp.int32, (NFRAMES, VOCAB), 1)
    toks = []
    for i in range(NCB):
        cbt = cbt_ref[i]
        cn = jnp.sum(cbt * cbt, axis=0, keepdims=True)
        s = jnp.dot(f[:, i * d : (i + 1) * d], cbt, precision=DP,
                    preferred_element_type=jnp.float32)
        scores = cn - 2.0 * s
        m = jnp.min(scores, axis=-1, keepdims=True)
        toks.append(jnp.min(jnp.where(scores == m, idx, VOCAB), axis=-1)
                    .astype(jnp.int32))
    out_ref[0] = jnp.stack(toks, axis=0)


def kernel(waveform, W1, b1, W2, b2, codebooks):
    B = waveform.shape[0]
    ge_np, go_np, tw_np, w_np, perm_np = _dft_constants()
    ge = jnp.asarray(ge_np)
    go = jnp.asarray(go_np)
    tw = jnp.asarray(tw_np)
    w = jnp.asarray(w_np)
    perm = jnp.asarray(perm_np)

    pad = N_FFT // 2
    xp = jnp.pad(waveform, ((0, 0), (pad, pad)), mode='reflect')
    xp = jnp.pad(xp, ((0, 0), (0, YROWS * HOP - xp.shape[1])))
    y = xp.reshape(B, YROWS, HOP)

    a1 = jnp.concatenate([W1[:, :, k].T for k in range(3)], axis=0)
    a2 = jnp.concatenate([W2[:, :, k].T for k in range(3)], axis=0)
    b1r = b1.reshape(1, -1)
    b2r = b2.reshape(1, -1)
    cbt = jnp.transpose(codebooks, (0, 2, 1))

    const = lambda shape: pl.BlockSpec(shape, lambda b: (0,) * len(shape))
    out = pl.pallas_call(
        _tokenizer_kernel,
        grid=(B,),
        in_specs=[
            pl.BlockSpec((1, YROWS, HOP), lambda b: (b, 0, 0)),
            const((NH, NH)),
            const((NH, NH)),
            const((2, NH // 2)),
            const((FPAD, N_MELS)),
            const((HOP, HOP)),
            const((384, 256)),
            const((1, 256)),
            const((768, 512)),
            const((1, 512)),
            const((NCB, DM // NCB, VOCAB)),
        ],
        out_specs=pl.BlockSpec((1, NCB, NFRAMES), lambda b: (b, 0, 0)),
        out_shape=jax.ShapeDtypeStruct((B, NCB, NFRAMES), jnp.int32),
    )(y, ge, go, tw, w, perm, a1, b1r, a2, b2r, cbt)
    return out


# speedup vs baseline: 3.1337x; 1.0759x over previous
import functools
import math

import jax
import jax.numpy as jnp
import numpy as np
from jax.experimental import pallas as pl

SR = 24000
N_FFT = 1024
HOP = 320
N_MELS = 128
VOCAB = 1024
NCB = 4
DM = 512
NFRAMES = 751
YROWS = 754
HP = jax.lax.Precision.HIGHEST
DP = jax.lax.Precision.DEFAULT


def _mel_fb_np():
    n_freqs = N_FFT // 2 + 1
    all_freqs = np.linspace(0.0, SR / 2.0, n_freqs)

    def hz_to_mel(f):
        return 2595.0 * np.log10(1.0 + f / 700.0)

    def mel_to_hz(m):
        return 700.0 * (10.0 ** (m / 2595.0) - 1.0)

    m_pts = np.linspace(hz_to_mel(0.0), hz_to_mel(SR / 2.0), N_MELS + 2)
    f_pts = mel_to_hz(m_pts)
    f_diff = f_pts[1:] - f_pts[:-1]
    slopes = f_pts[None, :] - all_freqs[:, None]
    down = -slopes[:, :-2] / f_diff[:-1]
    up = slopes[:, 2:] / f_diff[1:]
    return np.maximum(0.0, np.minimum(down, up))


NFREQ = N_FFT // 2 + 1
FPAD = 640
NH = N_FFT // 2
HHOP = HOP // 2
HYROWS = 754


@functools.lru_cache(maxsize=1)
def _dft_constants():
    n = np.arange(N_FFT)
    win = 0.5 - 0.5 * np.cos(2.0 * np.pi * n / N_FFT)
    m = np.arange(NH)
    f = np.arange(NH // 2 + 1)
    fs = np.arange(1, NH // 2)
    dft = np.concatenate(
        [np.cos(2.0 * np.pi * m[:, None] * f[None, :] / NH),
         np.sin(2.0 * np.pi * m[:, None] * fs[None, :] / NH)], axis=1
    )
    ge = win[0::2][:, None] * dft
    go = win[1::2][:, None] * dft
    fq = np.arange(NH // 2)
    tw = np.stack([np.cos(2.0 * np.pi * fq / N_FFT),
                   np.sin(2.0 * np.pi * fq / N_FFT)], axis=0)
    fb = _mel_fb_np()
    w = np.zeros((FPAD, N_MELS))
    w[0:256] = fb[0:256]
    w[256:512] = fb[512:256:-1]
    w[512] = fb[256]
    perm = np.zeros((HOP, HOP), np.float32)
    for mm in range(HHOP):
        perm[2 * mm, mm] = 1.0
        perm[2 * mm + 1, HHOP + mm] = 1.0
    return (np.asarray(ge, np.float32), np.asarray(go, np.float32),
            np.asarray(tw, np.float32), np.asarray(w, np.float32), perm)


def _frames_half(y):
    return jnp.concatenate(
        [y[0:NFRAMES], y[1 : NFRAMES + 1], y[2 : NFRAMES + 2],
         y[3 : NFRAMES + 3, : NH - 3 * HHOP]],
        axis=1,
    )


MINI_B = 1


def _tokenizer_kernel(y_ref, ge_ref, go_ref, tw_ref, w_ref, perm_ref, a1_ref,
                      b1_ref, a2_ref, b2_ref, cbt_ref, out_ref):
    for bb in range(MINI_B):
        _tokenizer_one(y_ref[bb], ge_ref, go_ref, tw_ref, w_ref, perm_ref,
                       a1_ref, b1_ref, a2_ref, b2_ref, cbt_ref, out_ref, bb)


def _tokenizer_one(y, ge_ref, go_ref, tw_ref, w_ref, perm_ref, a1_ref,
                   b1_ref, a2_ref, b2_ref, cbt_ref, out_ref, bb):
    z = jnp.dot(y, perm_ref[...], precision=HP,
                preferred_element_type=jnp.float32)
    ye = z[:, :HHOP]
    yo = z[:, HHOP:]
    ue = jnp.dot(_frames_half(ye), ge_ref[...], precision=HP,
                 preferred_element_type=jnp.float32)
    uo = jnp.dot(_frames_half(yo), go_ref[...], precision=HP,
                 preferred_element_type=jnp.float32)
    nq = NH // 2
    z1 = jnp.zeros((NFRAMES, 1), jnp.float32)
    re_e = ue[:, :nq]
    re_o = uo[:, :nq]
    s_e = jnp.concatenate([z1, ue[:, nq + 1 :]], axis=1)
    s_o = jnp.concatenate([z1, uo[:, nq + 1 :]], axis=1)
    c = tw_ref[0:1, :]
    s = tw_ref[1:2, :]
    re_t = c * re_o - s * s_o
    im_t = -(c * s_o + s * re_o)
    im_e = -s_e
    p_plus = (re_e + re_t) ** 2 + (im_e + im_t) ** 2
    p_minus = (re_e - re_t) ** 2 + (im_e - im_t) ** 2
    p256 = ue[:, nq : nq + 1] ** 2 + uo[:, nq : nq + 1] ** 2
    power = jnp.concatenate(
        [p_plus, p_minus, p256, jnp.zeros((NFRAMES, FPAD - 2 * nq - 1),
                                          jnp.float32)], axis=1)
    mel = jnp.dot(power, w_ref[...], precision=DP,
                  preferred_element_type=jnp.float32)
    mel = jnp.log(jnp.clip(mel, 1e-5, None))

    zc = jnp.zeros((1, N_MELS), jnp.float32)
    melp = jnp.concatenate([zc, mel, zc], axis=0)
    a1 = a1_ref[...]
    h = (jnp.dot(melp[0:NFRAMES], a1[0:128], precision=DP,
                 preferred_element_type=jnp.float32)
         + jnp.dot(melp[1 : NFRAMES + 1], a1[128:256], precision=DP,
                   preferred_element_type=jnp.float32)
         + jnp.dot(melp[2 : NFRAMES + 2], a1[256:384], precision=DP,
                   preferred_element_type=jnp.float32)
         + b1_ref[...])
    h = 0.5 * h * (1.0 + jax.lax.erf(h * (1.0 / math.sqrt(2.0))))

    zh = jnp.zeros((1, 256), jnp.float32)
    hp = jnp.concatenate([zh, h, zh], axis=0)
    a2 = a2_ref[...]
    f = (jnp.dot(hp[0:NFRAMES], a2[0:256], precision=DP,
                 preferred_element_type=jnp.float32)
         + jnp.dot(hp[1 : NFRAMES + 1], a2[256:512], precision=DP,
                   preferred_element_type=jnp.float32)
         + jnp.dot(hp[2 : NFRAMES + 2], a2[512:768], precision=DP,
                   preferred_element_type=jnp.float32)
         + b2_ref[...])

    d = DM // NCB
    toks = []
    for i in range(NCB):
        cbt2 = cbt_ref[i]
        cn = 0.25 * jnp.sum(cbt2 * cbt2, axis=0, keepdims=True)
        s2 = jnp.dot(f[:, i * d : (i + 1) * d], cbt2, precision=DP,
                     preferred_element_type=jnp.float32)
        scores = s2 + cn
        toks.append(jnp.argmin(scores, axis=-1).astype(jnp.int32))
    out_ref[bb] = jnp.stack(toks, axis=0)


def kernel(waveform, W1, b1, W2, b2, codebooks):
    B = waveform.shape[0]
    ge_np, go_np, tw_np, w_np, perm_np = _dft_constants()
    ge = jnp.asarray(ge_np)
    go = jnp.asarray(go_np)
    tw = jnp.asarray(tw_np)
    w = jnp.asarray(w_np)
    perm = jnp.asarray(perm_np)

    L = waveform.shape[1]
    xp = jnp.concatenate(
        [waveform[:, 512:0:-1], waveform,
         waveform[:, L - 2 : L - 514 : -1],
         jnp.zeros((B, YROWS * HOP - L - 1024), waveform.dtype)], axis=1)
    y = xp.reshape(B, YROWS, HOP)

    a1 = jnp.concatenate([W1[:, :, k].T for k in range(3)], axis=0)
    a2 = jnp.concatenate([W2[:, :, k].T for k in range(3)], axis=0)
    b1r = b1.reshape(1, -1)
    b2r = b2.reshape(1, -1)
    cbt = jnp.transpose(codebooks, (0, 2, 1)) * (-2.0)

    const = lambda shape: pl.BlockSpec(shape, lambda b: (0,) * len(shape))
    out = pl.pallas_call(
        _tokenizer_kernel,
        grid=(B // MINI_B,),
        in_specs=[
            pl.BlockSpec((MINI_B, YROWS, HOP), lambda b: (b, 0, 0)),
            const((NH, NH)),
            const((NH, NH)),
            const((2, NH // 2)),
            const((FPAD, N_MELS)),
            const((HOP, HOP)),
            const((384, 256)),
            const((1, 256)),
            const((768, 512)),
            const((1, 512)),
            const((NCB, DM // NCB, VOCAB)),
        ],
        out_specs=pl.BlockSpec((MINI_B, NCB, NFRAMES), lambda b: (b, 0, 0)),
        out_shape=jax.ShapeDtypeStruct((B, NCB, NFRAMES), jnp.int32),
    )(y, ge, go, tw, w, perm, a1, b1r, a2, b2r, cbt)
    return out
